# Initial kernel scaffold; baseline (speedup 1.0000x reference)
import jax
import jax.numpy as jnp
from jax.experimental import pallas as pl


def kernel(x, gate_w, expert_bias, w1, w2, w3, sw1, sw2, sw3):
    raise NotImplementedError("write your pallas kernel here")


# speedup vs baseline: 2.8124x; 2.8124x over previous
import functools

import jax
import jax.numpy as jnp
from jax.experimental import pallas as pl
from jax.experimental.pallas import tpu as pltpu

DIM = 1024
INTER = 512
E = 8
TOPK = 2
T = 2048

_NEG = -1e30


def _moe_body(x_ref, gw_ref, bias_ref, w1_ref, w2_ref, w3_ref,
              sw1_ref, sw2_ref, sw3_ref, out_ref, we_ref):
    e = pl.program_id(0)

    @pl.when(e == 0)
    def _gating_and_shared():
        x = x_ref[...]
        logits = jax.lax.dot_general(
            x, gw_ref[...], (((1,), (1,)), ((), ())),
            preferred_element_type=jnp.float32)
        m = jnp.max(logits, axis=1, keepdims=True)
        ex = jnp.exp(logits - m)
        scores = ex / jnp.sum(ex, axis=1, keepdims=True)
        scores_b = scores + bias_ref[...]
        lane = jax.lax.broadcasted_iota(jnp.int32, (T, E), 1)
        m1 = jnp.max(scores_b, axis=1, keepdims=True)
        a1 = jnp.min(jnp.where(scores_b == m1, lane, E), axis=1, keepdims=True)
        sb2 = jnp.where(lane == a1, _NEG, scores_b)
        m2 = jnp.max(sb2, axis=1, keepdims=True)
        a2 = jnp.min(jnp.where(sb2 == m2, lane, E), axis=1, keepdims=True)
        sel = (lane == a1) | (lane == a2)
        we_ref[...] = jnp.where(sel, scores, 0.0)
        h1 = jax.lax.dot_general(x, sw1_ref[...], (((1,), (1,)), ((), ())),
                                 preferred_element_type=jnp.float32)
        h3 = jax.lax.dot_general(x, sw3_ref[...], (((1,), (1,)), ((), ())),
                                 preferred_element_type=jnp.float32)
        h = (h1 * jax.lax.logistic(h1)) * h3
        out_ref[...] = jax.lax.dot_general(
            h, sw2_ref[...], (((1,), (1,)), ((), ())),
            preferred_element_type=jnp.float32)

    @pl.when(e > 0)
    def _expert():
        ee = e - 1
        x = x_ref[...]
        lane = jax.lax.broadcasted_iota(jnp.int32, (T, E), 1)
        wcol = jnp.sum(jnp.where(lane == ee, we_ref[...], 0.0),
                       axis=1, keepdims=True)
        h1 = jax.lax.dot_general(x, w1_ref[0], (((1,), (1,)), ((), ())),
                                 preferred_element_type=jnp.float32)
        h3 = jax.lax.dot_general(x, w3_ref[0], (((1,), (1,)), ((), ())),
                                 preferred_element_type=jnp.float32)
        h = (h1 * jax.lax.logistic(h1)) * h3
        o = jax.lax.dot_general(h, w2_ref[0], (((1,), (1,)), ((), ())),
                                preferred_element_type=jnp.float32)
        out_ref[...] += wcol * o


@functools.partial(jax.jit, static_argnames=())
def kernel(x, gate_w, expert_bias, w1, w2, w3, sw1, sw2, sw3):
    b, s, d = x.shape
    x2 = x.reshape(s, d)
    bias2 = expert_bias.reshape(1, E)
    out = pl.pallas_call(
        _moe_body,
        grid=(E + 1,),
        in_specs=[
            pl.BlockSpec((T, DIM), lambda e: (0, 0)),
            pl.BlockSpec((E, DIM), lambda e: (0, 0)),
            pl.BlockSpec((1, E), lambda e: (0, 0)),
            pl.BlockSpec((1, INTER, DIM),
                         lambda e: (jnp.maximum(e - 1, 0), 0, 0)),
            pl.BlockSpec((1, DIM, INTER),
                         lambda e: (jnp.maximum(e - 1, 0), 0, 0)),
            pl.BlockSpec((1, INTER, DIM),
                         lambda e: (jnp.maximum(e - 1, 0), 0, 0)),
            pl.BlockSpec((INTER, DIM), lambda e: (0, 0)),
            pl.BlockSpec((DIM, INTER), lambda e: (0, 0)),
            pl.BlockSpec((INTER, DIM), lambda e: (0, 0)),
        ],
        out_specs=pl.BlockSpec((T, DIM), lambda e: (0, 0)),
        out_shape=jax.ShapeDtypeStruct((T, DIM), jnp.float32),
        scratch_shapes=[pltpu.VMEM((T, E), jnp.float32)],
        compiler_params=pltpu.CompilerParams(
            dimension_semantics=("arbitrary",)),
    )(x2, gate_w, bias2, w1, w2, w3, sw1, sw2, sw3)
    return out.reshape(b, s, d)


# speedup vs baseline: 2.8192x; 1.0024x over previous
import functools

import jax
import jax.numpy as jnp
from jax.experimental import pallas as pl
from jax.experimental.pallas import tpu as pltpu

DIM = 1024
INTER = 512
E = 8
TOPK = 2
T = 2048

_NEG = -1e30


def _moe_body(x_ref, gw_ref, bias_ref, w1_ref, w2_ref, w3_ref,
              sw1_ref, sw2_ref, sw3_ref, out_ref, we_ref):
    e = pl.program_id(0)

    @pl.when(e == 0)
    def _gating_and_shared():
        x = x_ref[...]
        logits = jax.lax.dot_general(
            x, gw_ref[...], (((1,), (1,)), ((), ())),
            preferred_element_type=jnp.float32)
        m = jnp.max(logits, axis=1, keepdims=True)
        ex = jnp.exp(logits - m)
        scores = ex / jnp.sum(ex, axis=1, keepdims=True)
        scores_b = scores + bias_ref[...]
        lane = jax.lax.broadcasted_iota(jnp.int32, (T, E), 1)
        m1 = jnp.max(scores_b, axis=1, keepdims=True)
        a1 = jnp.min(jnp.where(scores_b == m1, lane, E), axis=1, keepdims=True)
        sb2 = jnp.where(lane == a1, _NEG, scores_b)
        m2 = jnp.max(sb2, axis=1, keepdims=True)
        a2 = jnp.min(jnp.where(sb2 == m2, lane, E), axis=1, keepdims=True)
        sel = (lane == a1) | (lane == a2)
        we_ref[...] = jnp.where(sel, scores, 0.0)
        xb = x.astype(jnp.bfloat16)
        h1 = jax.lax.dot_general(xb, sw1_ref[...].astype(jnp.bfloat16),
                                 (((1,), (1,)), ((), ())),
                                 preferred_element_type=jnp.float32)
        h3 = jax.lax.dot_general(xb, sw3_ref[...].astype(jnp.bfloat16),
                                 (((1,), (1,)), ((), ())),
                                 preferred_element_type=jnp.float32)
        h = (h1 * jax.lax.logistic(h1)) * h3
        out_ref[...] = jax.lax.dot_general(
            h.astype(jnp.bfloat16), sw2_ref[...].astype(jnp.bfloat16),
            (((1,), (1,)), ((), ())),
            preferred_element_type=jnp.float32)

    @pl.when(e > 0)
    def _expert():
        ee = e - 1
        xb = x_ref[...].astype(jnp.bfloat16)
        lane = jax.lax.broadcasted_iota(jnp.int32, (T, E), 1)
        wcol = jnp.sum(jnp.where(lane == ee, we_ref[...], 0.0),
                       axis=1, keepdims=True)
        h1 = jax.lax.dot_general(xb, w1_ref[0].astype(jnp.bfloat16),
                                 (((1,), (1,)), ((), ())),
                                 preferred_element_type=jnp.float32)
        h3 = jax.lax.dot_general(xb, w3_ref[0].astype(jnp.bfloat16),
                                 (((1,), (1,)), ((), ())),
                                 preferred_element_type=jnp.float32)
        h = (h1 * jax.lax.logistic(h1)) * h3
        o = jax.lax.dot_general(h.astype(jnp.bfloat16),
                                w2_ref[0].astype(jnp.bfloat16),
                                (((1,), (1,)), ((), ())),
                                preferred_element_type=jnp.float32)
        out_ref[...] += wcol * o


@functools.partial(jax.jit, static_argnames=())
def kernel(x, gate_w, expert_bias, w1, w2, w3, sw1, sw2, sw3):
    b, s, d = x.shape
    x2 = x.reshape(s, d)
    bias2 = expert_bias.reshape(1, E)
    out = pl.pallas_call(
        _moe_body,
        grid=(E + 1,),
        in_specs=[
            pl.BlockSpec((T, DIM), lambda e: (0, 0)),
            pl.BlockSpec((E, DIM), lambda e: (0, 0)),
            pl.BlockSpec((1, E), lambda e: (0, 0)),
            pl.BlockSpec((1, INTER, DIM),
                         lambda e: (jnp.maximum(e - 1, 0), 0, 0)),
            pl.BlockSpec((1, DIM, INTER),
                         lambda e: (jnp.maximum(e - 1, 0), 0, 0)),
            pl.BlockSpec((1, INTER, DIM),
                         lambda e: (jnp.maximum(e - 1, 0), 0, 0)),
            pl.BlockSpec((INTER, DIM), lambda e: (0, 0)),
            pl.BlockSpec((DIM, INTER), lambda e: (0, 0)),
            pl.BlockSpec((INTER, DIM), lambda e: (0, 0)),
        ],
        out_specs=pl.BlockSpec((T, DIM), lambda e: (0, 0)),
        out_shape=jax.ShapeDtypeStruct((T, DIM), jnp.float32),
        scratch_shapes=[pltpu.VMEM((T, E), jnp.float32)],
        compiler_params=pltpu.CompilerParams(
            dimension_semantics=("arbitrary",)),
    )(x2, gate_w, bias2, w1, w2, w3, sw1, sw2, sw3)
    return out.reshape(b, s, d)
